# Initial kernel scaffold; baseline (speedup 1.0000x reference)
import jax
import jax.numpy as jnp
from jax.experimental import pallas as pl


def kernel(logits, labels):
    raise NotImplementedError("write your pallas kernel here")


# speedup vs baseline: 1.2114x; 1.2114x over previous
import functools

import jax
import jax.numpy as jnp
from jax.experimental import pallas as pl
from jax.experimental.pallas import tpu as pltpu

NBINS = 15
PAD_BINS = 16


def _soft_ece_kernel(logits_ref, labels_ref, out_ref, acc_ref, *, nblocks):
    i = pl.program_id(0)

    @pl.when(i == 0)
    def _init():
        acc_ref[...] = jnp.zeros_like(acc_ref)

    x = logits_ref[...]
    b, c = x.shape
    m = jnp.max(x, axis=1, keepdims=True)
    e = jnp.exp(x - m)
    s = jnp.sum(e, axis=1, keepdims=True)

    lab = labels_ref[0]
    col = jax.lax.broadcasted_iota(jnp.int32, (b, c), 1)
    e_lab = jnp.sum(jnp.where(col == lab, e, 0.0), axis=1, keepdims=True)

    max_prob = 1.0 / s
    pred_prob = e_lab / s

    bin_width = jnp.float32(1.0 / NBINS)
    bins = jnp.floor(max_prob / bin_width).astype(jnp.int32)
    bins = jnp.clip(bins, 0, NBINS - 1)

    bin_iota = jax.lax.broadcasted_iota(jnp.int32, (b, PAD_BINS), 1)
    onehot = (bins == bin_iota).astype(jnp.float32)

    acc_ref[0:1, :] += jnp.sum(onehot, axis=0, keepdims=True)
    acc_ref[1:2, :] += jnp.sum(onehot * max_prob, axis=0, keepdims=True)
    acc_ref[2:3, :] += jnp.sum(onehot * pred_prob, axis=0, keepdims=True)

    @pl.when(i == nblocks - 1)
    def _finish():
        counts = acc_ref[0:1, :]
        conf_sum = acc_ref[1:2, :]
        acc_sum = acc_ref[2:3, :]
        safe = jnp.maximum(counts, 1.0)
        conf_mean = jnp.where(counts > 0, conf_sum / safe, 0.0)
        acc_mean = jnp.where(counts > 0, acc_sum / safe, 0.0)
        num = jnp.sum(counts * jnp.abs(conf_mean - acc_mean), keepdims=True)
        den = jnp.sum(counts, keepdims=True)
        out_ref[...] = num / den


def kernel(logits, labels):
    n, c = logits.shape
    block = 256
    nblocks = n // block
    labels3 = labels.astype(jnp.int32).reshape(nblocks, block, 1)

    out = pl.pallas_call(
        functools.partial(_soft_ece_kernel, nblocks=nblocks),
        grid=(nblocks,),
        in_specs=[
            pl.BlockSpec((block, c), lambda i: (i, 0)),
            pl.BlockSpec((1, block, 1), lambda i: (i, 0, 0)),
        ],
        out_specs=pl.BlockSpec((1, 1), lambda i: (0, 0)),
        out_shape=jax.ShapeDtypeStruct((1, 1), jnp.float32),
        scratch_shapes=[pltpu.VMEM((3, PAD_BINS), jnp.float32)],
    )(logits, labels3)
    return out[0, 0]


# speedup vs baseline: 1.3896x; 1.1472x over previous
import functools

import jax
import jax.numpy as jnp
from jax.experimental import pallas as pl
from jax.experimental.pallas import tpu as pltpu

NBINS = 15
PAD_BINS = 16


def _soft_ece_kernel(logits_ref, labels_ref, out_ref, acc_ref, *, nblocks):
    i = pl.program_id(0)

    @pl.when(i == 0)
    def _init():
        acc_ref[...] = jnp.zeros_like(acc_ref)

    x = logits_ref[...]
    b, c = x.shape
    m = jnp.max(x, axis=1, keepdims=True)
    e = jnp.exp(x - m)
    s = jnp.sum(e, axis=1, keepdims=True)

    lab = labels_ref[0]
    col = jax.lax.broadcasted_iota(jnp.int32, (b, c), 1)
    e_lab = jnp.sum(jnp.where(col == lab, e, 0.0), axis=1, keepdims=True)

    max_prob = 1.0 / s
    pred_prob = e_lab / s

    bin_width = jnp.float32(1.0 / NBINS)
    bins = jnp.floor(max_prob / bin_width).astype(jnp.int32)
    bins = jnp.clip(bins, 0, NBINS - 1)

    bin_iota = jax.lax.broadcasted_iota(jnp.int32, (b, PAD_BINS), 1)
    onehot = (bins == bin_iota).astype(jnp.float32)

    acc_ref[0:1, :] += jnp.sum(onehot, axis=0, keepdims=True)
    acc_ref[1:2, :] += jnp.sum(onehot * max_prob, axis=0, keepdims=True)
    acc_ref[2:3, :] += jnp.sum(onehot * pred_prob, axis=0, keepdims=True)

    @pl.when(i == nblocks - 1)
    def _finish():
        counts = acc_ref[0:1, :]
        conf_sum = acc_ref[1:2, :]
        acc_sum = acc_ref[2:3, :]
        safe = jnp.maximum(counts, 1.0)
        conf_mean = jnp.where(counts > 0, conf_sum / safe, 0.0)
        acc_mean = jnp.where(counts > 0, acc_sum / safe, 0.0)
        num = jnp.sum(counts * jnp.abs(conf_mean - acc_mean), keepdims=True)
        den = jnp.sum(counts, keepdims=True)
        out_ref[...] = num / den


def kernel(logits, labels):
    n, c = logits.shape
    block = 512
    nblocks = n // block
    labels3 = labels.astype(jnp.int32).reshape(nblocks, block, 1)

    out = pl.pallas_call(
        functools.partial(_soft_ece_kernel, nblocks=nblocks),
        grid=(nblocks,),
        in_specs=[
            pl.BlockSpec((block, c), lambda i: (i, 0)),
            pl.BlockSpec((1, block, 1), lambda i: (i, 0, 0)),
        ],
        out_specs=pl.BlockSpec((1, 1), lambda i: (0, 0)),
        out_shape=jax.ShapeDtypeStruct((1, 1), jnp.float32),
        scratch_shapes=[pltpu.VMEM((3, PAD_BINS), jnp.float32)],
    )(logits, labels3)
    return out[0, 0]


# speedup vs baseline: 1.5116x; 1.0878x over previous
import functools

import jax
import jax.numpy as jnp
from jax.experimental import pallas as pl
from jax.experimental.pallas import tpu as pltpu

NBINS = 15
PAD_BINS = 16


def _soft_ece_kernel(logits_ref, labels_ref, out_ref, acc_ref, *, nblocks):
    i = pl.program_id(0)

    @pl.when(i == 0)
    def _init():
        acc_ref[...] = jnp.zeros_like(acc_ref)

    x = logits_ref[...]
    b, c = x.shape
    m = jnp.max(x, axis=1, keepdims=True)
    e = jnp.exp(x - m)
    s = jnp.sum(e, axis=1, keepdims=True)

    lab = labels_ref[0]
    col = jax.lax.broadcasted_iota(jnp.int32, (b, c), 1)
    e_lab = jnp.sum(jnp.where(col == lab, e, 0.0), axis=1, keepdims=True)

    max_prob = 1.0 / s
    pred_prob = e_lab / s

    bin_width = jnp.float32(1.0 / NBINS)
    bins = jnp.floor(max_prob / bin_width).astype(jnp.int32)
    bins = jnp.clip(bins, 0, NBINS - 1)

    bin_iota = jax.lax.broadcasted_iota(jnp.int32, (b, PAD_BINS), 1)
    onehot = (bins == bin_iota).astype(jnp.float32)

    acc_ref[0:1, :] += jnp.sum(onehot, axis=0, keepdims=True)
    acc_ref[1:2, :] += jnp.sum(onehot * max_prob, axis=0, keepdims=True)
    acc_ref[2:3, :] += jnp.sum(onehot * pred_prob, axis=0, keepdims=True)

    @pl.when(i == nblocks - 1)
    def _finish():
        counts = acc_ref[0:1, :]
        conf_sum = acc_ref[1:2, :]
        acc_sum = acc_ref[2:3, :]
        safe = jnp.maximum(counts, 1.0)
        conf_mean = jnp.where(counts > 0, conf_sum / safe, 0.0)
        acc_mean = jnp.where(counts > 0, acc_sum / safe, 0.0)
        num = jnp.sum(counts * jnp.abs(conf_mean - acc_mean), keepdims=True)
        den = jnp.sum(counts, keepdims=True)
        out_ref[...] = num / den


def kernel(logits, labels):
    n, c = logits.shape
    block = 1024
    nblocks = n // block
    labels3 = labels.astype(jnp.int32).reshape(nblocks, block, 1)

    out = pl.pallas_call(
        functools.partial(_soft_ece_kernel, nblocks=nblocks),
        grid=(nblocks,),
        in_specs=[
            pl.BlockSpec((block, c), lambda i: (i, 0)),
            pl.BlockSpec((1, block, 1), lambda i: (i, 0, 0)),
        ],
        out_specs=pl.BlockSpec((1, 1), lambda i: (0, 0)),
        out_shape=jax.ShapeDtypeStruct((1, 1), jnp.float32),
        scratch_shapes=[pltpu.VMEM((3, PAD_BINS), jnp.float32)],
    )(logits, labels3)
    return out[0, 0]


# speedup vs baseline: 1.5574x; 1.0303x over previous
import functools

import jax
import jax.numpy as jnp
from jax.experimental import pallas as pl
from jax.experimental.pallas import tpu as pltpu

NBINS = 15
PAD_BINS = 16


def _soft_ece_kernel(logits_ref, labels_ref, out_ref, acc_ref, *, nblocks):
    i = pl.program_id(0)

    @pl.when(i == 0)
    def _init():
        acc_ref[...] = jnp.zeros_like(acc_ref)

    x = logits_ref[...]
    b, c = x.shape
    m = jnp.max(x, axis=1, keepdims=True)
    e = jnp.exp(x - m)
    s = jnp.sum(e, axis=1, keepdims=True)

    lab = labels_ref[0]
    col = jax.lax.broadcasted_iota(jnp.int32, (b, c), 1)
    e_lab = jnp.sum(jnp.where(col == lab, e, 0.0), axis=1, keepdims=True)

    max_prob = 1.0 / s
    pred_prob = e_lab / s

    bin_width = jnp.float32(1.0 / NBINS)
    bins = jnp.floor(max_prob / bin_width).astype(jnp.int32)
    bins = jnp.clip(bins, 0, NBINS - 1)

    bin_iota = jax.lax.broadcasted_iota(jnp.int32, (b, PAD_BINS), 1)
    onehot = (bins == bin_iota).astype(jnp.float32)

    acc_ref[0:1, :] += jnp.sum(onehot, axis=0, keepdims=True)
    acc_ref[1:2, :] += jnp.sum(onehot * max_prob, axis=0, keepdims=True)
    acc_ref[2:3, :] += jnp.sum(onehot * pred_prob, axis=0, keepdims=True)

    @pl.when(i == nblocks - 1)
    def _finish():
        counts = acc_ref[0:1, :]
        conf_sum = acc_ref[1:2, :]
        acc_sum = acc_ref[2:3, :]
        safe = jnp.maximum(counts, 1.0)
        conf_mean = jnp.where(counts > 0, conf_sum / safe, 0.0)
        acc_mean = jnp.where(counts > 0, acc_sum / safe, 0.0)
        num = jnp.sum(counts * jnp.abs(conf_mean - acc_mean), keepdims=True)
        den = jnp.sum(counts, keepdims=True)
        out_ref[...] = num / den


def kernel(logits, labels):
    n, c = logits.shape
    block = 2048
    nblocks = n // block
    labels3 = labels.astype(jnp.int32).reshape(nblocks, block, 1)

    out = pl.pallas_call(
        functools.partial(_soft_ece_kernel, nblocks=nblocks),
        grid=(nblocks,),
        in_specs=[
            pl.BlockSpec((block, c), lambda i: (i, 0)),
            pl.BlockSpec((1, block, 1), lambda i: (i, 0, 0)),
        ],
        out_specs=pl.BlockSpec((1, 1), lambda i: (0, 0)),
        out_shape=jax.ShapeDtypeStruct((1, 1), jnp.float32),
        scratch_shapes=[pltpu.VMEM((3, PAD_BINS), jnp.float32)],
    )(logits, labels3)
    return out[0, 0]
